# baseline (speedup 1.0000x reference)
import jax
import jax.numpy as jnp
from jax import lax
from jax.experimental import pallas as pl
from jax.experimental.pallas import tpu as pltpu
from jax.experimental.pallas import tpu_sc as plsc

BATCH = 1024
VOCAB = 100000
S = 128
L = 16
NC, NS = 2, 16
NW = NC * NS
RPW = BATCH // NW

CH = 128
NCHUNK = RPW * S // CH
CPS = 4
STG = NCHUNK // CPS
JPC = CH // RPW
JPS = CPS * JPC


def _acc_neg_logsigmoid(x, acc):
    u = jnp.exp(-jnp.abs(x))
    w = u / (u + 2.0)
    w2 = w * w
    poly = 1.0 + w2 * (0.33333334 + w2 * (0.2 + w2 * (0.14285715 + w2 * 0.11111111)))
    logsig = jnp.minimum(x, 0.0) - 2.0 * w * poly
    return acc - logsig


def _phys(col, bpart):
    return (
        lax.shift_left(lax.shift_right_logical(col, 3), 13)
        + lax.shift_left(col & 7, 7)
        + bpart
    )


def _sc_body(inp_hbm, tgt_hbm, smp_hbm, out_hbm,
             tgt_i, smp_gi, smp_si, tgt_s, smp_s, part_v,
             id_sems, sc_sems, tsem):
    c = lax.axis_index("c")
    s = lax.axis_index("s")
    wid = s * NC + c
    base = wid * RPW

    iota = lax.broadcasted_iota(jnp.int32, (L,), 0)
    row0 = base + iota
    row1 = base + L + iota
    bpart0 = lax.shift_left(lax.shift_right_logical(row0, 7), 10) + (row0 & 127)
    bpart1 = lax.shift_left(lax.shift_right_logical(row1, 7), 10) + (row1 & 127)

    def build(j, carry):
        smp_gi[pl.ds(j * RPW, L)] = row0 * S + j
        smp_gi[pl.ds(j * RPW + L, L)] = row1 * S + j
        return carry

    def addoff(j, carry):
        smp_si[pl.ds(j * RPW, L)] = _phys(smp_si[pl.ds(j * RPW, L)], bpart0)
        smp_si[pl.ds(j * RPW + L, L)] = _phys(smp_si[pl.ds(j * RPW + L, L)], bpart1)
        return carry

    def loss(j, acc):
        a0 = _acc_neg_logsigmoid(t0 - smp_s[pl.ds(j * RPW, L)], acc[0])
        a1 = _acc_neg_logsigmoid(t1 - smp_s[pl.ds(j * RPW + L, L)], acc[1])
        return (a0, a1)

    id_cps = []
    for st in range(STG):
        lax.fori_loop(st * JPS, (st + 1) * JPS, build, 0, unroll=4)
        for k in range(CPS):
            g = st * CPS + k
            id_cps.append(pltpu.async_copy(
                smp_hbm.at[smp_gi.at[pl.ds(g * CH, CH)]],
                smp_si.at[pl.ds(g * CH, CH)], id_sems.at[st]))

    pltpu.sync_copy(tgt_hbm.at[pl.ds(base, RPW)], tgt_i)
    tgt_i[pl.ds(0, L)] = _phys(tgt_i[pl.ds(0, L)], bpart0)
    tgt_i[pl.ds(L, L)] = _phys(tgt_i[pl.ds(L, L)], bpart1)
    cp_t = pltpu.async_copy(inp_hbm.at[tgt_i], tgt_s, tsem)

    sc_cps = []
    for st in range(STG):
        for k in range(CPS):
            id_cps[st * CPS + k].wait()
        lax.fori_loop(st * JPS, (st + 1) * JPS, addoff, 0, unroll=4)
        for k in range(CPS):
            g = st * CPS + k
            sc_cps.append(pltpu.async_copy(
                inp_hbm.at[smp_si.at[pl.ds(g * CH, CH)]],
                smp_s.at[pl.ds(g * CH, CH)], sc_sems.at[st]))

    cp_t.wait()
    t0 = tgt_s[pl.ds(0, L)]
    t1 = tgt_s[pl.ds(L, L)]

    acc = (jnp.zeros((L,), jnp.float32), jnp.zeros((L,), jnp.float32))
    for st in range(STG):
        for k in range(CPS):
            sc_cps[st * CPS + k].wait()
        acc = lax.fori_loop(st * JPS, (st + 1) * JPS, loss, acc, unroll=4)

    part_v[...] = (acc[0] + acc[1]) * (1.0 / (BATCH * S))
    pltpu.sync_copy(part_v, out_hbm.at[wid])


@jax.jit
def _bpr_loss_sc(inp_flat, tgt, smp_flat):
    mesh = plsc.VectorSubcoreMesh(core_axis_name="c", subcore_axis_name="s")
    f = pl.kernel(
        _sc_body,
        out_type=jax.ShapeDtypeStruct((NW, L), jnp.float32),
        mesh=mesh,
        scratch_types=[
            pltpu.VMEM((RPW,), jnp.int32),
            pltpu.VMEM((RPW * S,), jnp.int32),
            pltpu.VMEM((RPW * S,), jnp.int32),
            pltpu.VMEM((RPW,), jnp.float32),
            pltpu.VMEM((RPW * S,), jnp.float32),
            pltpu.VMEM((L,), jnp.float32),
            pltpu.SemaphoreType.DMA((STG,)),
            pltpu.SemaphoreType.DMA((STG,)),
            pltpu.SemaphoreType.DMA,
        ],
    )
    return f(inp_flat, tgt, smp_flat)


def kernel(input, target, samples):
    flat = (
        jnp.swapaxes(input, 0, 1)
        .reshape(VOCAB // 8, 8, BATCH // 128, 128)
        .transpose(0, 2, 1, 3)
        .reshape(-1)
    )
    tgt = target.astype(jnp.int32)
    smp_flat = samples.astype(jnp.int32).reshape(-1)
    partials = _bpr_loss_sc(flat, tgt, smp_flat)
    return jnp.sum(partials)


# speedup vs baseline: 1.3352x; 1.3352x over previous
import jax
import jax.numpy as jnp
from jax import lax
from jax.experimental import pallas as pl
from jax.experimental.pallas import tpu as pltpu
from jax.experimental.pallas import tpu_sc as plsc

BATCH = 1024
VOCAB = 100000
S = 128
L = 16
NC, NS = 2, 16
NW = NC * NS
RPW = BATCH // NW

CH = 128
NCHUNK = RPW * S // CH


def _acc_neg_logsigmoid(x, acc):
    u = jnp.exp(-jnp.abs(x))
    w = u / (u + 2.0)
    w2 = w * w
    poly = 1.0 + w2 * (0.33333334 + w2 * (0.2 + w2 * (0.14285715 + w2 * 0.11111111)))
    logsig = jnp.minimum(x, 0.0) - 2.0 * w * poly
    return acc - logsig


def _phys(col, bpart):
    return (
        lax.shift_left(lax.shift_right_logical(col, 3), 13)
        + lax.shift_left(col & 7, 7)
        + bpart
    )


def _sc_body(inp_hbm, tgt_hbm, smp_hbm, out_hbm,
             tgt_i, smp_si, tgt_s, smp_s, part_v, sem, tsem):
    c = lax.axis_index("c")
    s = lax.axis_index("s")
    wid = s * NC + c
    base = wid * RPW

    iota = lax.broadcasted_iota(jnp.int32, (L,), 0)
    row0 = base + iota
    row1 = base + L + iota
    bpart0 = lax.shift_left(lax.shift_right_logical(row0, 7), 10) + (row0 & 127)
    bpart1 = lax.shift_left(lax.shift_right_logical(row1, 7), 10) + (row1 & 127)

    pltpu.sync_copy(smp_hbm.at[pl.ds(wid * RPW * S, RPW * S)], smp_si)

    pltpu.sync_copy(tgt_hbm.at[pl.ds(base, RPW)], tgt_i)
    tgt_i[pl.ds(0, L)] = _phys(tgt_i[pl.ds(0, L)], bpart0)
    tgt_i[pl.ds(L, L)] = _phys(tgt_i[pl.ds(L, L)], bpart1)
    cp_t = pltpu.async_copy(inp_hbm.at[tgt_i], tgt_s, tsem)

    def addoff(j, carry):
        smp_si[pl.ds(j * RPW, L)] = _phys(smp_si[pl.ds(j * RPW, L)], bpart0)
        smp_si[pl.ds(j * RPW + L, L)] = _phys(smp_si[pl.ds(j * RPW + L, L)], bpart1)
        return carry
    lax.fori_loop(0, S, addoff, 0)

    cps = [
        pltpu.async_copy(inp_hbm.at[smp_si.at[pl.ds(g * CH, CH)]],
                         smp_s.at[pl.ds(g * CH, CH)], sem)
        for g in range(NCHUNK)
    ]
    for cp in cps:
        cp.wait()
    cp_t.wait()

    t0 = tgt_s[pl.ds(0, L)]
    t1 = tgt_s[pl.ds(L, L)]

    def loss(j, acc):
        a0 = _acc_neg_logsigmoid(t0 - smp_s[pl.ds(j * RPW, L)], acc[0])
        a1 = _acc_neg_logsigmoid(t1 - smp_s[pl.ds(j * RPW + L, L)], acc[1])
        return (a0, a1)
    zero = jnp.zeros((L,), jnp.float32)
    acc0, acc1 = lax.fori_loop(0, S, loss, (zero, zero))

    part_v[...] = (acc0 + acc1) * (1.0 / (BATCH * S))
    pltpu.sync_copy(part_v, out_hbm.at[wid])


@jax.jit
def _bpr_loss_sc(inp_flat, tgt, smp_flat):
    mesh = plsc.VectorSubcoreMesh(core_axis_name="c", subcore_axis_name="s")
    f = pl.kernel(
        _sc_body,
        out_type=jax.ShapeDtypeStruct((NW, L), jnp.float32),
        mesh=mesh,
        scratch_types=[
            pltpu.VMEM((RPW,), jnp.int32),
            pltpu.VMEM((RPW * S,), jnp.int32),
            pltpu.VMEM((RPW,), jnp.float32),
            pltpu.VMEM((RPW * S,), jnp.float32),
            pltpu.VMEM((L,), jnp.float32),
            pltpu.SemaphoreType.DMA,
            pltpu.SemaphoreType.DMA,
        ],
    )
    return f(inp_flat, tgt, smp_flat)


def kernel(input, target, samples):
    flat = (
        jnp.swapaxes(input, 0, 1)
        .reshape(VOCAB // 8, 8, BATCH // 128, 128)
        .transpose(0, 2, 1, 3)
        .reshape(-1)
    )
    tgt = target.astype(jnp.int32)
    smp_t = (
        samples.astype(jnp.int32)
        .reshape(NW, RPW, S)
        .transpose(0, 2, 1)
        .reshape(-1)
    )
    partials = _bpr_loss_sc(flat, tgt, smp_t)
    return jnp.sum(partials)


# speedup vs baseline: 1.3405x; 1.0040x over previous
import jax
import jax.numpy as jnp
from jax import lax
from jax.experimental import pallas as pl
from jax.experimental.pallas import tpu as pltpu
from jax.experimental.pallas import tpu_sc as plsc

BATCH = 1024
VOCAB = 100000
S = 128
L = 16
NC, NS = 2, 16
NW = NC * NS
RPW = BATCH // NW

CH = 128
NCHUNK = RPW * S // CH


def _acc_neg_logsigmoid(x, acc):
    u = jnp.exp(-jnp.abs(x))
    w = u / (u + 2.0)
    w2 = w * w
    poly = 1.0 + w2 * (0.33333334 + w2 * (0.2 + w2 * (0.14285715 + w2 * 0.11111111)))
    logsig = jnp.minimum(x, 0.0) - 2.0 * w * poly
    return acc - logsig


def _phys(col, bpart):
    return (
        lax.shift_left(lax.shift_right_logical(col, 3), 13)
        + lax.shift_left(col & 7, 7)
        + bpart
    )


def _sc_body(inp_hbm, tgt_hbm, smp_hbm, out_hbm,
             tgt_i, smp_si, tgt_s, smp_s, part_v, sem, tsem):
    c = lax.axis_index("c")
    s = lax.axis_index("s")
    wid = s * NC + c
    base = wid * RPW

    iota = lax.broadcasted_iota(jnp.int32, (L,), 0)
    row0 = base + iota
    row1 = base + L + iota
    bpart0 = lax.shift_left(lax.shift_right_logical(row0, 7), 10) + (row0 & 127)
    bpart1 = lax.shift_left(lax.shift_right_logical(row1, 7), 10) + (row1 & 127)

    pltpu.sync_copy(smp_hbm.at[wid], smp_si)

    pltpu.sync_copy(tgt_hbm.at[pl.ds(base, RPW)], tgt_i)
    tgt_i[pl.ds(0, L)] = _phys(tgt_i[pl.ds(0, L)], bpart0)
    tgt_i[pl.ds(L, L)] = _phys(tgt_i[pl.ds(L, L)], bpart1)
    cp_t = pltpu.async_copy(inp_hbm.at[tgt_i], tgt_s, tsem)

    def addoff(j, carry):
        smp_si[pl.ds(j * RPW, L)] = _phys(smp_si[pl.ds(j * RPW, L)], bpart0)
        smp_si[pl.ds(j * RPW + L, L)] = _phys(smp_si[pl.ds(j * RPW + L, L)], bpart1)
        return carry

    BLK = 4
    CPB = NCHUNK // BLK
    JPB = S // BLK
    cps = []
    for blk in range(BLK):
        lax.fori_loop(blk * JPB, (blk + 1) * JPB, addoff, 0)
        for k in range(CPB):
            g = blk * CPB + k
            cps.append(pltpu.async_copy(
                inp_hbm.at[smp_si.at[pl.ds(g * CH, CH)]],
                smp_s.at[pl.ds(g * CH, CH)], sem))
    for cp in cps:
        cp.wait()
    cp_t.wait()

    t0 = tgt_s[pl.ds(0, L)]
    t1 = tgt_s[pl.ds(L, L)]

    def loss(j, acc):
        a0 = _acc_neg_logsigmoid(t0 - smp_s[pl.ds(j * RPW, L)], acc[0])
        a1 = _acc_neg_logsigmoid(t1 - smp_s[pl.ds(j * RPW + L, L)], acc[1])
        return (a0, a1)
    zero = jnp.zeros((L,), jnp.float32)
    acc0, acc1 = lax.fori_loop(0, S, loss, (zero, zero))

    part_v[...] = (acc0 + acc1) * (1.0 / (BATCH * S))
    pltpu.sync_copy(part_v, out_hbm.at[wid])


@jax.jit
def _bpr_loss_sc(inp_flat, tgt, smp_flat):
    mesh = plsc.VectorSubcoreMesh(core_axis_name="c", subcore_axis_name="s")
    f = pl.kernel(
        _sc_body,
        out_type=jax.ShapeDtypeStruct((NW, L), jnp.float32),
        mesh=mesh,
        scratch_types=[
            pltpu.VMEM((RPW,), jnp.int32),
            pltpu.VMEM((RPW * S,), jnp.int32),
            pltpu.VMEM((RPW,), jnp.float32),
            pltpu.VMEM((RPW * S,), jnp.float32),
            pltpu.VMEM((L,), jnp.float32),
            pltpu.SemaphoreType.DMA,
            pltpu.SemaphoreType.DMA,
        ],
    )
    return f(inp_flat, tgt, smp_flat)


def kernel(input, target, samples):
    flat = (
        jnp.swapaxes(input, 0, 1)
        .reshape(VOCAB // 8, 8, BATCH // 128, 128)
        .transpose(0, 2, 1, 3)
        .reshape(-1)
    )
    tgt = target.astype(jnp.int32)
    smp_t = (
        samples.astype(jnp.int32)
        .reshape(NW, RPW, S)
        .transpose(0, 2, 1)
        .reshape(NW, RPW * S)
    )
    partials = _bpr_loss_sc(flat, tgt, smp_t)
    return jnp.sum(partials)


# speedup vs baseline: 1.3506x; 1.0075x over previous
import jax
import jax.numpy as jnp
from jax import lax
from jax.experimental import pallas as pl
from jax.experimental.pallas import tpu as pltpu
from jax.experimental.pallas import tpu_sc as plsc

BATCH = 1024
VOCAB = 100000
S = 128
L = 16
NC, NS = 2, 16
NW = NC * NS
RPW = BATCH // NW

CH = 128
NCHUNK = RPW * S // CH


def _acc_neg_logsigmoid(x, acc):
    u = jnp.exp(-jnp.abs(x))
    w = u / (u + 2.0)
    w2 = w * w
    poly = 1.0 + w2 * (0.33333334 + w2 * (0.2 + w2 * (0.14285715 + w2 * 0.11111111)))
    logsig = jnp.minimum(x, 0.0) - 2.0 * w * poly
    return acc - logsig


def _phys(col, bpart):
    return (
        lax.shift_left(lax.shift_right_logical(col, 3), 13)
        + lax.shift_left(col & 7, 7)
        + bpart
    )


def _sc_body(inp_hbm, tgt_hbm, smp_hbm, out_hbm,
             tgt_i, smp_si, tgt_s, smp_s, part_v, sem, tsem):
    c = lax.axis_index("c")
    s = lax.axis_index("s")
    wid = s * NC + c
    base = wid * RPW

    iota = lax.broadcasted_iota(jnp.int32, (L,), 0)
    row0 = base + iota
    row1 = base + L + iota
    bpart0 = lax.shift_left(lax.shift_right_logical(row0, 7), 10) + (row0 & 127)
    bpart1 = lax.shift_left(lax.shift_right_logical(row1, 7), 10) + (row1 & 127)

    pltpu.sync_copy(smp_hbm.at[wid], smp_si)

    pltpu.sync_copy(tgt_hbm.at[pl.ds(base, RPW)], tgt_i)
    tgt_i[pl.ds(0, L)] = _phys(tgt_i[pl.ds(0, L)], bpart0)
    tgt_i[pl.ds(L, L)] = _phys(tgt_i[pl.ds(L, L)], bpart1)
    cp_t = pltpu.async_copy(inp_hbm.at[tgt_i], tgt_s, tsem)

    def addoff(j, carry):
        smp_si[pl.ds(j * RPW, L)] = _phys(smp_si[pl.ds(j * RPW, L)], bpart0)
        smp_si[pl.ds(j * RPW + L, L)] = _phys(smp_si[pl.ds(j * RPW + L, L)], bpart1)
        return carry

    BLK = 4
    CPB = NCHUNK // BLK
    JPB = S // BLK
    cps = []
    for blk in range(BLK):
        lax.fori_loop(blk * JPB, (blk + 1) * JPB, addoff, 0)
        for k in range(CPB):
            g = blk * CPB + k
            cps.append(pltpu.async_copy(
                inp_hbm.at[smp_si.at[pl.ds(g * CH, CH)]],
                smp_s.at[pl.ds(g * CH, CH)], sem.at[blk]))
    cp_t.wait()

    t0 = tgt_s[pl.ds(0, L)]
    t1 = tgt_s[pl.ds(L, L)]

    def loss(j, acc):
        a0 = _acc_neg_logsigmoid(t0 - smp_s[pl.ds(j * RPW, L)], acc[0])
        a1 = _acc_neg_logsigmoid(t1 - smp_s[pl.ds(j * RPW + L, L)], acc[1])
        return (a0, a1)
    zero = jnp.zeros((L,), jnp.float32)
    acc = (zero, zero)
    for blk in range(BLK):
        for k in range(CPB):
            cps[blk * CPB + k].wait()
        acc = lax.fori_loop(blk * JPB, (blk + 1) * JPB, loss, acc)
    acc0, acc1 = acc

    part_v[...] = (acc0 + acc1) * (1.0 / (BATCH * S))
    pltpu.sync_copy(part_v, out_hbm.at[wid])


@jax.jit
def _bpr_loss_sc(inp_flat, tgt, smp_flat):
    mesh = plsc.VectorSubcoreMesh(core_axis_name="c", subcore_axis_name="s")
    f = pl.kernel(
        _sc_body,
        out_type=jax.ShapeDtypeStruct((NW, L), jnp.float32),
        mesh=mesh,
        scratch_types=[
            pltpu.VMEM((RPW,), jnp.int32),
            pltpu.VMEM((RPW * S,), jnp.int32),
            pltpu.VMEM((RPW,), jnp.float32),
            pltpu.VMEM((RPW * S,), jnp.float32),
            pltpu.VMEM((L,), jnp.float32),
            pltpu.SemaphoreType.DMA((4,)),
            pltpu.SemaphoreType.DMA,
        ],
    )
    return f(inp_flat, tgt, smp_flat)


def kernel(input, target, samples):
    flat = (
        jnp.swapaxes(input, 0, 1)
        .reshape(VOCAB // 8, 8, BATCH // 128, 128)
        .transpose(0, 2, 1, 3)
        .reshape(-1)
    )
    tgt = target.astype(jnp.int32)
    smp_t = (
        samples.astype(jnp.int32)
        .reshape(NW, RPW, S)
        .transpose(0, 2, 1)
        .reshape(NW, RPW * S)
    )
    partials = _bpr_loss_sc(flat, tgt, smp_t)
    return jnp.sum(partials)
